# baseline (speedup 1.0000x reference)
import functools

import jax
import jax.numpy as jnp
from jax.experimental import pallas as pl
from jax.experimental.pallas import tpu as pltpu


def _gru_kernel(x_ref, emb_ref, W_ih_ref, W_hh_ref, b_comb_ref, b_hhn_ref,
                W_fc_ref, b_fc_ref, out_ref, h_ref, G_ref, Whh_ref, Wfc_ref,
                *, H, U):
    t = pl.program_id(0)
    B = h_ref.shape[0]
    V = G_ref.shape[0]

    @pl.when(t == 0)
    def _init():
        Whh_ref[...] = W_hh_ref[...].astype(jnp.bfloat16).T
        Wfc_ref[...] = W_fc_ref[...].astype(jnp.bfloat16).T
        G_ref[...] = (
            jax.lax.dot_general(emb_ref[...], W_ih_ref[...],
                                (((1,), (1,)), ((), ())),
                                preferred_element_type=jnp.float32)
            + b_comb_ref[...]
        ).astype(jnp.bfloat16)
        h_ref[...] = jnp.zeros_like(h_ref)

    h = h_ref[...]
    hs = []
    for u in range(U):
        idx = x_ref[t * U + u]
        onehot = (idx[:, None]
                  == jax.lax.broadcasted_iota(jnp.int32, (B, V), 1)
                  ).astype(jnp.bfloat16)
        gi = jnp.dot(onehot, G_ref[...], preferred_element_type=jnp.float32)

        gh = jnp.dot(h.astype(jnp.bfloat16), Whh_ref[...],
                     preferred_element_type=jnp.float32)

        r = jax.nn.sigmoid(gi[:, :H] + gh[:, :H])
        z = jax.nn.sigmoid(gi[:, H:2 * H] + gh[:, H:2 * H])
        n = jnp.tanh(gi[:, 2 * H:] + r * (gh[:, 2 * H:] + b_hhn_ref[...]))
        h = (1.0 - z) * n + z * h

        hs.append(h.astype(jnp.bfloat16))
    h_ref[...] = h

    logitsU = (jnp.dot(jnp.concatenate(hs, axis=0), Wfc_ref[...],
                       preferred_element_type=jnp.float32)
               + b_fc_ref[...])
    out_ref[...] = jnp.swapaxes(logitsU.reshape(U, B, V), 0, 1)


def kernel(x_in, emb, W_ih, W_hh, b_ih, b_hh, W_fc, b_fc):
    B, S = x_in.shape
    V, E = emb.shape
    H = W_hh.shape[1]

    x = x_in.astype(jnp.int32).T
    b_comb = (b_ih + jnp.concatenate(
        [b_hh[:2 * H], jnp.zeros_like(b_hh[2 * H:])])).reshape(1, -1)
    b_hhn = b_hh[2 * H:].reshape(1, -1)
    b_fc2 = b_fc.reshape(1, -1)

    U = 16
    full = lambda shape: pl.BlockSpec(shape, lambda t: (0,) * len(shape))
    out = pl.pallas_call(
        functools.partial(_gru_kernel, H=H, U=U),
        grid=(S // U,),
        in_specs=[
            full((S, B)),
            full((V, E)),
            full((3 * H, E)),
            full((3 * H, H)),
            full((1, 3 * H)),
            full((1, H)),
            full((V, H)),
            full((1, V)),
        ],
        out_specs=pl.BlockSpec((B, U, V), lambda t: (0, t, 0)),
        out_shape=jax.ShapeDtypeStruct((B, S, V), jnp.float32),
        scratch_shapes=[
            pltpu.VMEM((B, H), jnp.float32),
            pltpu.VMEM((V, 3 * H), jnp.bfloat16),
            pltpu.VMEM((H, 3 * H), jnp.bfloat16),
            pltpu.VMEM((H, V), jnp.bfloat16),
        ],
    )(x, emb, W_ih, W_hh, b_comb, b_hhn, W_fc, b_fc2)
    return out


# speedup vs baseline: 1.5156x; 1.5156x over previous
import functools

import jax
import jax.numpy as jnp
from jax.experimental import pallas as pl
from jax.experimental.pallas import tpu as pltpu


def _gru_kernel(x_ref, emb_ref, W_ih_ref, W_hh_ref, b_comb_ref, b_hhn_ref,
                W_fc_ref, b_fc_ref, out_ref, h_ref, G_ref, Whh_ref, Wfc_ref,
                *, H, U):
    t = pl.program_id(0)
    B = h_ref.shape[0]
    V = G_ref.shape[0]

    @pl.when(t == 0)
    def _init():
        Whh_ref[...] = W_hh_ref[...].astype(jnp.bfloat16).T
        Wfc_ref[...] = W_fc_ref[...].astype(jnp.bfloat16).T
        G_ref[...] = (
            jax.lax.dot_general(emb_ref[...], W_ih_ref[...],
                                (((1,), (1,)), ((), ())),
                                preferred_element_type=jnp.float32)
            + b_comb_ref[...]
        ).astype(jnp.bfloat16)
        h_ref[...] = jnp.zeros_like(h_ref)

    h = h_ref[...]
    hs = []
    for u in range(U):
        idx = x_ref[t * U + u]
        onehot = (idx[:, None]
                  == jax.lax.broadcasted_iota(jnp.int32, (B, V), 1)
                  ).astype(jnp.bfloat16)
        gi = jnp.dot(onehot, G_ref[...], preferred_element_type=jnp.float32)

        gh = jnp.dot(h.astype(jnp.bfloat16), Whh_ref[...],
                     preferred_element_type=jnp.float32)

        r = jax.nn.sigmoid(gi[:, :H] + gh[:, :H])
        z = jax.nn.sigmoid(gi[:, H:2 * H] + gh[:, H:2 * H])
        n = jnp.tanh(gi[:, 2 * H:] + r * (gh[:, 2 * H:] + b_hhn_ref[...]))
        h = n + z * (h - n)

        hs.append(h.astype(jnp.bfloat16))
    h_ref[...] = h

    logitsU = (jnp.dot(jnp.concatenate(hs, axis=0), Wfc_ref[...],
                       preferred_element_type=jnp.float32)
               + b_fc_ref[...])
    out_ref[...] = jnp.swapaxes(logitsU.reshape(U, B, V), 0, 1)


def kernel(x_in, emb, W_ih, W_hh, b_ih, b_hh, W_fc, b_fc):
    B, S = x_in.shape
    V, E = emb.shape
    H = W_hh.shape[1]

    x = x_in.astype(jnp.int32).T
    b_comb = (b_ih + jnp.concatenate(
        [b_hh[:2 * H], jnp.zeros_like(b_hh[2 * H:])])).reshape(1, -1)
    b_hhn = b_hh[2 * H:].reshape(1, -1)
    b_fc2 = b_fc.reshape(1, -1)

    U = 8
    full = lambda shape: pl.BlockSpec(shape, lambda t: (0,) * len(shape))
    out = pl.pallas_call(
        functools.partial(_gru_kernel, H=H, U=U),
        grid=(S // U,),
        in_specs=[
            full((S, B)),
            full((V, E)),
            full((3 * H, E)),
            full((3 * H, H)),
            full((1, 3 * H)),
            full((1, H)),
            full((V, H)),
            full((1, V)),
        ],
        out_specs=pl.BlockSpec((B, U, V), lambda t: (0, t, 0)),
        out_shape=jax.ShapeDtypeStruct((B, S, V), jnp.float32),
        scratch_shapes=[
            pltpu.VMEM((B, H), jnp.float32),
            pltpu.VMEM((V, 3 * H), jnp.bfloat16),
            pltpu.VMEM((H, 3 * H), jnp.bfloat16),
            pltpu.VMEM((H, V), jnp.bfloat16),
        ],
    )(x, emb, W_ih, W_hh, b_comb, b_hhn, W_fc, b_fc2)
    return out
